# baseline (speedup 1.0000x reference)
import functools

import jax
import jax.numpy as jnp
import numpy as np
from jax import lax
from jax.experimental import pallas as pl
from jax.experimental.pallas import tpu as pltpu
from jax.experimental.pallas import tpu_sc as plsc

_FIELD_DIMS = [40000] * 26
_OFFSETS = np.array((0, *np.cumsum(_FIELD_DIMS)[:-1]), dtype=np.int32)

_B = 16384
_F = 26
_V = sum(_FIELD_DIMS)
_VBIG = (_V // 1024) * 1024
_FD = _FIELD_DIMS[0]
_FDPAD = (_FD + 1023) // 1024 * 1024
_NC = 2
_NS = 16
_NW = _NC * _NS
_BPW = _B // _NW
_L = 16
_NG = 26
_FPG = _F // _NG
_EPG = _FPG * _BPW


def _make_sc_kernel():
  mesh = plsc.VectorSubcoreMesh(
      core_axis_name="c", subcore_axis_name="s",
      num_cores=_NC, num_subcores=_NS)

  @functools.partial(
      pl.kernel,
      mesh=mesh,
      compiler_params=pltpu.CompilerParams(needs_layout_passes=False),
      out_type=jax.ShapeDtypeStruct((_B,), jnp.float32),
      scratch_types=[
          pltpu.VMEM((_F * _BPW,), jnp.int32),
          pltpu.VMEM((_F * _BPW,), jnp.float32),
          pltpu.VMEM((_BPW,), jnp.float32),
          pltpu.VMEM((1,), jnp.float32),
          pltpu.SemaphoreType.DMA,
          pltpu.SemaphoreType.DMA,
      ],
  )
  def sc_kernel(xt_hbm, table_hbm, tlast_hbm, bias_hbm, out_hbm,
                idx_v, vals_v, acc_v, bias_v, sem, semi):
    wid = lax.axis_index("s") * _NC + lax.axis_index("c")
    pltpu.sync_copy(bias_hbm, bias_v)
    idx_copies = [
        pltpu.async_copy(
            xt_hbm.at[f, pl.ds(wid * _BPW, _BPW)],
            idx_v.at[pl.ds(f * _BPW, _BPW)], semi)
        for f in range(_F)
    ]
    for c in idx_copies:
      c.wait()
    copies = [
        pltpu.async_copy(
            (table_hbm.at[pl.ds(int(_OFFSETS[f]), _FD)] if f < _F - 1
             else tlast_hbm.at[pl.ds(0, _FD)])
            .at[idx_v.at[pl.ds(f * _BPW, _BPW)]],
            vals_v.at[pl.ds(f * _BPW, _BPW)], sem)
        for f in range(_F)
    ]
    bias = plsc.load_gather(bias_v, [jnp.zeros((_L,), jnp.int32)])
    for g in range(_NG):
      copies[g].wait()
      for rc in range(_BPW // _L):
        acc = bias if g == 0 else acc_v[pl.ds(rc * _L, _L)]
        for f in range(g * _FPG, (g + 1) * _FPG):
          acc = acc + vals_v[pl.ds(f * _BPW + rc * _L, _L)]
        acc_v[pl.ds(rc * _L, _L)] = acc
    pltpu.sync_copy(acc_v, out_hbm.at[pl.ds(wid * _BPW, _BPW)])

  return sc_kernel


_SC_KERNEL = _make_sc_kernel()


def kernel(x, table, bias):
  idx = x.T.astype(jnp.int32)
  tbig = table[:_VBIG].reshape(-1)
  tlast = lax.pad(table[_OFFSETS[-1]:], jnp.float32(0),
                  ((0, _FDPAD - _FD, 0), (0, 0, 0))).reshape(-1)
  sums = _SC_KERNEL(idx, tbig, tlast, bias)
  return sums[:, None]


# speedup vs baseline: 1.0373x; 1.0373x over previous
import functools

import jax
import jax.numpy as jnp
import numpy as np
from jax import lax
from jax.experimental import pallas as pl
from jax.experimental.pallas import tpu as pltpu
from jax.experimental.pallas import tpu_sc as plsc

_FIELD_DIMS = [40000] * 26
_OFFSETS = np.array((0, *np.cumsum(_FIELD_DIMS)[:-1]), dtype=np.int32)

_B = 16384
_F = 26
_V = sum(_FIELD_DIMS)
_VPAD = (_V + 1023) // 1024 * 1024
_NC = 2
_NS = 16
_NW = _NC * _NS
_BPW = _B // _NW
_L = 16
_NG = 26
_FPG = _F // _NG
_EPG = _FPG * _BPW


def _make_sc_kernel():
  mesh = plsc.VectorSubcoreMesh(
      core_axis_name="c", subcore_axis_name="s",
      num_cores=_NC, num_subcores=_NS)

  @functools.partial(
      pl.kernel,
      mesh=mesh,
      compiler_params=pltpu.CompilerParams(needs_layout_passes=False),
      out_type=jax.ShapeDtypeStruct((_B,), jnp.float32),
      scratch_types=[
          pltpu.VMEM((_F * _BPW,), jnp.int32),
          pltpu.VMEM((_F * _BPW,), jnp.float32),
          pltpu.VMEM((_BPW,), jnp.float32),
          pltpu.VMEM((1,), jnp.float32),
          pltpu.SemaphoreType.DMA,
          pltpu.SemaphoreType.DMA,
      ],
  )
  def sc_kernel(xt_hbm, table_hbm, bias_hbm, out_hbm,
                idx_v, vals_v, acc_v, bias_v, sem, semi):
    wid = lax.axis_index("s") * _NC + lax.axis_index("c")
    pltpu.sync_copy(bias_hbm, bias_v)
    idx_copies = [
        pltpu.async_copy(
            xt_hbm.at[f, pl.ds(wid * _BPW, _BPW)],
            idx_v.at[pl.ds(f * _BPW, _BPW)], semi)
        for f in range(_F)
    ]
    copies = []
    for f in range(_F):
      idx_copies[f].wait()
      copies.append(pltpu.async_copy(
          table_hbm.at[pl.ds(int(_OFFSETS[f]), _FIELD_DIMS[f])]
                   .at[idx_v.at[pl.ds(f * _BPW, _BPW)]],
          vals_v.at[pl.ds(f * _BPW, _BPW)], sem))
    bias = plsc.load_gather(bias_v, [jnp.zeros((_L,), jnp.int32)])
    for g in range(_NG):
      copies[g].wait()
      for rc in range(_BPW // _L):
        acc = bias if g == 0 else acc_v[pl.ds(rc * _L, _L)]
        for f in range(g * _FPG, (g + 1) * _FPG):
          acc = acc + vals_v[pl.ds(f * _BPW + rc * _L, _L)]
        acc_v[pl.ds(rc * _L, _L)] = acc
    pltpu.sync_copy(acc_v, out_hbm.at[pl.ds(wid * _BPW, _BPW)])

  return sc_kernel


_SC_KERNEL = _make_sc_kernel()


def kernel(x, table, bias):
  idx = x.T.astype(jnp.int32)
  tpad = lax.pad(table, jnp.float32(0), ((0, _VPAD - _V, 0), (0, 0, 0)))
  sums = _SC_KERNEL(idx, tpad.reshape(-1), bias)
  return sums[:, None]
